# Initial kernel scaffold; baseline (speedup 1.0000x reference)
import jax
import jax.numpy as jnp
from jax.experimental import pallas as pl


def kernel(step, lprobs, scores, original_batch_idxs):
    raise NotImplementedError("write your pallas kernel here")


# speedup vs baseline: 9.3959x; 9.3959x over previous
import jax
import jax.numpy as jnp
from jax.experimental import pallas as pl
from jax.experimental.pallas import tpu as pltpu

BSZ = 32
BEAM = 8
VOCAB = 100000
ROWS = BSZ * BEAM
WIN = 1024
NW = (VOCAB + WIN - 1) // WIN
NSEL = 8
GROUPS = 4
MINI = BEAM // GROUPS
DIVERSITY = -0.5
NEG = float('-inf')
IBIG = 2**30


def _k1_window_topk(x_ref, wid_ref):
    parts = []
    for w in range(NW):
        lo = w * WIN
        hi = min(VOCAB, lo + WIN)
        parts.append(jnp.max(x_ref[:, lo:hi], axis=1, keepdims=True))
    bm = jnp.concatenate(parts, axis=1)
    wiota = jax.lax.broadcasted_iota(jnp.int32, (8, NW), 1)
    picks = []
    for _ in range(NSEL):
        m = jnp.max(bm, axis=1, keepdims=True)
        cand = jnp.where(bm == m, wiota, jnp.int32(NW))
        w = jnp.min(cand, axis=1, keepdims=True)
        picks.append(w)
        bm = jnp.where(wiota == w, NEG, bm)
    wid_ref[...] = jnp.concatenate(picks, axis=1)


def _k2_select(wids_ref, bias_ref, *refs):
    i = pl.program_id(0)
    wins = refs[: BEAM * NSEL]
    sc_ref, ix_ref, bm_ref = refs[BEAM * NSEL:]
    liota = jax.lax.broadcasted_iota(jnp.int32, (1, WIN), 1)
    rows_v, rows_g = [], []
    for r in range(BEAM):
        vs, gs = [], []
        for j in range(NSEL):
            wid = wids_ref[i * BEAM + r, j]
            piece = wins[r * NSEL + j][r:r + 1, :]
            g = wid * WIN + liota
            vs.append(jnp.where(g < VOCAB, piece, NEG))
            gs.append(g)
        rows_v.append(jnp.concatenate(vs, axis=1))
        rows_g.append(jnp.concatenate(gs, axis=1))
    cv = jnp.concatenate(rows_v, axis=0)
    gi = jnp.concatenate(rows_g, axis=0)

    vals, idxs = [], []
    for _ in range(NSEL):
        m = jnp.max(cv, axis=1, keepdims=True)
        ix = jnp.min(jnp.where(cv == m, gi, IBIG), axis=1, keepdims=True)
        vals.append(m)
        idxs.append(ix)
        cv = jnp.where(gi == ix, NEG, cv)
    v8 = jnp.concatenate(vals, axis=1)
    i8 = jnp.concatenate(idxs, axis=1)

    bias = bias_ref[:, 0:1]
    jiota = jax.lax.broadcasted_iota(jnp.int32, (MINI, NSEL), 0)
    outv = [None] * BEAM
    outi = [None] * BEAM
    outb = [None] * BEAM
    prev = []
    for g in range(GROUPS):
        v2 = jnp.concatenate([v8[g:g + 1], v8[g + GROUPS:g + GROUPS + 1]], 0)
        i2 = jnp.concatenate([i8[g:g + 1], i8[g + GROUPS:g + GROUPS + 1]], 0)
        b2 = jnp.concatenate([bias[g:g + 1], bias[g + GROUPS:g + GROUPS + 1]], 0)
        s2 = v2 + b2
        for p in prev:
            s2 = s2 + DIVERSITY * (i2 == p).astype(jnp.float32)
        fk = jiota * VOCAB + i2
        for rank in range(MINI):
            m1 = jnp.max(s2, axis=1, keepdims=True)
            m = jnp.max(m1, axis=0, keepdims=True)
            km = jnp.where(s2 == m, fk, IBIG)
            k1 = jnp.min(km, axis=1, keepdims=True)
            k = jnp.min(k1, axis=0, keepdims=True)
            vsel = k % VOCAB
            jsel = k // VOCAB
            col = rank * GROUPS + g
            outv[col] = m
            outi[col] = vsel
            outb[col] = jsel * GROUPS + g
            prev.append(vsel)
            s2 = jnp.where(fk == k, NEG, s2)
    sc_ref[...] = jnp.concatenate(outv, axis=1)[None]
    ix_ref[...] = jnp.concatenate(outi, axis=1)[None]
    bm_ref[...] = jnp.concatenate(outb, axis=1)[None]


def kernel(step, lprobs, scores, original_batch_idxs):
    del original_batch_idxs
    x2d = lprobs.reshape(ROWS, VOCAB)
    bias = jax.lax.dynamic_index_in_dim(scores, step - 1, axis=2,
                                        keepdims=False)
    bias2d = jnp.tile(bias.reshape(ROWS, 1), (1, 8))

    wids = pl.pallas_call(
        _k1_window_topk,
        grid=(BSZ,),
        in_specs=[pl.BlockSpec((BEAM, VOCAB), lambda i: (i, 0))],
        out_specs=pl.BlockSpec((BEAM, NSEL), lambda i: (i, 0)),
        out_shape=jax.ShapeDtypeStruct((ROWS, NSEL), jnp.int32),
    )(x2d)

    def win_spec(r, j):
        def imap(i, w_ref):
            return (i, w_ref[i * BEAM + r, j])
        return pl.BlockSpec((BEAM, WIN), imap)

    grid_spec = pltpu.PrefetchScalarGridSpec(
        num_scalar_prefetch=1,
        grid=(BSZ,),
        in_specs=[pl.BlockSpec((BEAM, 8), lambda i, w: (i, 0))] +
                 [win_spec(r, j) for r in range(BEAM) for j in range(NSEL)],
        out_specs=[pl.BlockSpec((1, 1, BEAM), lambda i, w: (i, 0, 0))] * 3,
    )
    sc3, ix3, bm3 = pl.pallas_call(
        _k2_select,
        grid_spec=grid_spec,
        out_shape=[
            jax.ShapeDtypeStruct((BSZ, 1, BEAM), jnp.float32),
            jax.ShapeDtypeStruct((BSZ, 1, BEAM), jnp.int32),
            jax.ShapeDtypeStruct((BSZ, 1, BEAM), jnp.int32),
        ],
    )(wids, bias2d, *([x2d] * (BEAM * NSEL)))
    return (sc3.reshape(BSZ, BEAM), ix3.reshape(BSZ, BEAM),
            bm3.reshape(BSZ, BEAM))
